# baseline (speedup 1.0000x reference)
import jax
import jax.numpy as jnp
from jax import lax
from jax.experimental import pallas as pl
from jax.experimental.pallas import tpu as pltpu

_N, _C, _H, _W = 8, 96, 224, 224
_CB = 24
_R = _CB * _H


def _block_mat(b):
    i = lax.broadcasted_iota(jnp.int32, (_W, _W), 0)
    j = lax.broadcasted_iota(jnp.int32, (_W, _W), 1)
    return (i // b == j // b).astype(jnp.float32)


def _roll0(x, k):
    n = x.shape[0]
    return pltpu.roll(x, k % n, 0)


def _row_block_sum(xf, b, mh):
    t = xf + jnp.where(mh & 1 == 0, _roll0(xf, -1), _roll0(xf, 1))
    if b == 4:
        t = t + jnp.where(mh < 2, _roll0(t, -2), _roll0(t, 2))
    return t


def _pooled(x_ref, o_ref, b):
    xf = x_ref[...].reshape(_R, _W)
    mh = lax.broadcasted_iota(jnp.int32, (_R, 1), 0) & (b - 1)
    t = _row_block_sum(xf, b, mh)
    hi = t.astype(jnp.bfloat16)
    lo = (t - hi.astype(jnp.float32)).astype(jnp.bfloat16)
    a = _block_mat(b).astype(jnp.bfloat16)
    u = (jnp.dot(hi, a, preferred_element_type=jnp.float32)
         + jnp.dot(lo, a, preferred_element_type=jnp.float32))
    o_ref[...] = jnp.where(u > 0, xf, 0.0).reshape(_CB, _H, _W)


def _body(x_ref, o_ref):
    g = pl.program_id(1)

    @pl.when(g == 0)
    def _():
        o_ref[...] = x_ref[...]

    @pl.when(g == 1)
    def _():
        o_ref[...] = jnp.maximum(x_ref[...], 0.0)

    @pl.when(g == 2)
    def _():
        _pooled(x_ref, o_ref, 2)

    @pl.when(g == 3)
    def _():
        _pooled(x_ref, o_ref, 4)


def kernel(activation):
    return pl.pallas_call(
        _body,
        grid=(_N, _C // _CB),
        in_specs=[pl.BlockSpec((None, _CB, _H, _W),
                               lambda n, c: (n, c, 0, 0))],
        out_specs=pl.BlockSpec((None, _CB, _H, _W),
                               lambda n, c: (n, c, 0, 0)),
        out_shape=jax.ShapeDtypeStruct((_N, _C, _H, _W), jnp.float32),
        compiler_params=pltpu.CompilerParams(
            dimension_semantics=("parallel", "parallel")),
    )(activation)
